# baseline (speedup 1.0000x reference)
import functools

import jax
import jax.numpy as jnp
from jax import lax
from jax.experimental import pallas as pl
from jax.experimental.pallas import tpu as pltpu
from jax.experimental.pallas import tpu_sc as plsc

N_CORES = 2
N_SUBCORES = 16
K = 128
NB = 10
DEG_W = 8


def _sc_params():
    return pltpu.CompilerParams(use_tc_tiling_on_sc=False,
                                skip_device_barrier=True)


def _make_agg(n_pad, d_feat, n_groups):
    mesh = plsc.VectorSubcoreMesh(core_axis_name="c", subcore_axis_name="s")
    rows_per_tile = n_pad // N_SUBCORES
    d_half = d_feat // N_CORES

    @functools.partial(
        pl.kernel,
        out_type=jax.ShapeDtypeStruct((n_pad, d_feat), jnp.float32),
        mesh=mesh,
        compiler_params=_sc_params(),
        scratch_types=[
            pltpu.VMEM((2, NB, 2, K), jnp.int32),
            pltpu.VMEM((2, NB, K, d_half), jnp.float32),
            pltpu.VMEM_SHARED((n_pad, d_half), jnp.float32),
            pltpu.VMEM_SHARED((n_pad, d_half), jnp.float32),
            pltpu.SemaphoreType.DMA,
            pltpu.SemaphoreType.DMA,
            pltpu.SemaphoreType.DMA,
        ],
    )
    def agg(y_hbm, idx_hbm, zeros_hbm, out_hbm, idx_v, rows_v, acc_sh, y_sh,
            gsem, isem, ssem):
        cid = lax.axis_index("c")
        sid = lax.axis_index("s")
        r0 = sid * rows_per_tile
        col0 = cid * d_half
        pltpu.sync_copy(zeros_hbm.at[pl.ds(r0, rows_per_tile)],
                        acc_sh.at[pl.ds(r0, rows_per_tile)])
        pltpu.sync_copy(y_hbm.at[pl.ds(r0, rows_per_tile), pl.ds(col0, d_half)],
                        y_sh.at[pl.ds(r0, rows_per_tile)])
        pltpu.sync_copy(idx_hbm.at[sid, 0], idx_v.at[0])
        plsc.subcore_barrier()

        def scatter_descs(parity):
            return [
                pltpu.make_async_copy(rows_v.at[parity, b],
                                      acc_sh.at[idx_v.at[parity, b, 1]],
                                      ssem)
                for b in range(NB)
            ]

        def group_body(parity, g):
            nxt = g + 1
            descs = [
                pltpu.async_copy(y_sh.at[idx_v.at[parity, b, 0]],
                                 rows_v.at[parity, b], gsem)
                for b in range(NB)
            ]

            @pl.when(g > 0)
            def _drain_prev_scatters():
                for d in scatter_descs(1 - parity):
                    d.wait()

            @pl.when(nxt < n_groups)
            def _prefetch():
                pltpu.async_copy(idx_hbm.at[sid, nxt], idx_v.at[1 - parity], isem)

            for b in range(NB):
                descs[b].wait()
                pltpu.async_copy(rows_v.at[parity, b],
                                 acc_sh.at[idx_v.at[parity, b, 1]], ssem,
                                 add=True)

            @pl.when(nxt < n_groups)
            def _drain_prefetch():
                pltpu.make_async_copy(idx_hbm.at[sid, nxt],
                                      idx_v.at[1 - parity], isem).wait()

        def body(gg, carry):
            group_body(0, 2 * gg)
            group_body(1, 2 * gg + 1)
            return carry

        lax.fori_loop(0, n_groups // 2, body, 0)
        for d in scatter_descs(1):
            d.wait()
        plsc.subcore_barrier()
        pltpu.sync_copy(acc_sh.at[pl.ds(r0, rows_per_tile)],
                        out_hbm.at[pl.ds(r0, rows_per_tile), pl.ds(col0, d_half)])

    return agg


def _make_deg(n_pad, n_groups):
    mesh = plsc.VectorSubcoreMesh(core_axis_name="c", subcore_axis_name="s")
    rows_per_tile = n_pad // N_SUBCORES
    half_groups = n_groups // 2

    @functools.partial(
        pl.kernel,
        out_type=jax.ShapeDtypeStruct((N_CORES, n_pad, DEG_W), jnp.float32),
        mesh=mesh,
        compiler_params=_sc_params(),
        scratch_types=[
            pltpu.VMEM((2, NB, 2, K), jnp.int32),
            pltpu.VMEM((K, DEG_W), jnp.float32),
            pltpu.VMEM_SHARED((n_pad, DEG_W), jnp.float32),
            pltpu.SemaphoreType.DMA,
            pltpu.SemaphoreType.DMA,
        ],
    )
    def deg(ones_hbm, idx_hbm, zeros_hbm, out_hbm, idx_v, ones_v, acc_sh, isem,
            ssem):
        cid = lax.axis_index("c")
        sid = lax.axis_index("s")
        g0 = cid * half_groups
        r0 = sid * rows_per_tile
        pltpu.sync_copy(zeros_hbm.at[pl.ds(r0, rows_per_tile)],
                        acc_sh.at[pl.ds(r0, rows_per_tile)])
        pltpu.sync_copy(ones_hbm, ones_v)
        pltpu.sync_copy(idx_hbm.at[sid, g0], idx_v.at[0])
        plsc.subcore_barrier()

        def scatter_descs(parity):
            return [
                pltpu.make_async_copy(ones_v, acc_sh.at[idx_v.at[parity, b, 1]],
                                      ssem)
                for b in range(NB)
            ]

        def group_body(parity, g):
            nxt = g + 1

            @pl.when(g > 0)
            def _drain_prev_scatters():
                for d in scatter_descs(1 - parity):
                    d.wait()

            @pl.when(nxt < half_groups)
            def _prefetch():
                pltpu.async_copy(idx_hbm.at[sid, g0 + nxt], idx_v.at[1 - parity],
                                 isem)

            for b in range(NB):
                pltpu.async_copy(ones_v, acc_sh.at[idx_v.at[parity, b, 1]],
                                 ssem, add=True)

            @pl.when(nxt < half_groups)
            def _drain_prefetch():
                pltpu.make_async_copy(idx_hbm.at[sid, g0 + nxt],
                                      idx_v.at[1 - parity], isem).wait()

        def body(gg, carry):
            group_body(0, 2 * gg)
            group_body(1, 2 * gg + 1)
            return carry

        lax.fori_loop(0, half_groups // 2, body, 0)
        for d in scatter_descs(1):
            d.wait()
        plsc.subcore_barrier()
        pltpu.sync_copy(acc_sh.at[pl.ds(r0, rows_per_tile)],
                        out_hbm.at[cid, pl.ds(r0, rows_per_tile)])

    return deg


def _tc_call(body, out_shapes):
    return pl.pallas_call(body, out_shape=out_shapes)


def _tc1(c0_ref, c1_ref, x_ref, w1_ref, y1_ref, dinv_ref):
    n = x_ref.shape[0]
    deg = c0_ref[:, :1] + c1_ref[:, :1] + 1.0
    dinv = lax.rsqrt(deg)
    dinv_ref[...] = dinv
    h = jnp.dot(x_ref[...], w1_ref[...], precision=lax.Precision.HIGHEST,
                preferred_element_type=jnp.float32)
    y1_ref[:n] = h * dinv
    y1_ref[n:] = jnp.zeros_like(y1_ref[n:])


def _tc_mid(s_ref, y_ref, dinv_ref, b_ref, w_ref, out_ref):
    n = dinv_ref.shape[0]
    dinv = dinv_ref[...]
    p = dinv * (s_ref[:n] + y_ref[:n]) + b_ref[...]
    h = jnp.maximum(p, 0.0)
    out_ref[:n] = jnp.dot(h, w_ref[...], precision=lax.Precision.HIGHEST,
                          preferred_element_type=jnp.float32) * dinv
    out_ref[n:] = jnp.zeros_like(out_ref[n:])


def _tc_last(s_ref, y_ref, dinv_ref, b_ref, out_ref):
    n, c = out_ref.shape
    z = dinv_ref[...] * (s_ref[:n, :c] + y_ref[:n, :c]) + b_ref[...]
    z = z - jnp.max(z, axis=1, keepdims=True)
    out_ref[...] = z - jnp.log(jnp.sum(jnp.exp(z), axis=1, keepdims=True))


@jax.jit
def kernel(x, edge_index, W1, b1, W2, b2, W3, b3):
    n = x.shape[0]
    e = edge_index.shape[1]
    n_classes = W3.shape[1]
    hidden = W1.shape[1]

    n_pad = ((n + 1 + 8 * N_SUBCORES - 1) // (8 * N_SUBCORES)) * (8 * N_SUBCORES)
    group_e = N_SUBCORES * K * NB
    n_groups = (e + group_e - 1) // group_e
    n_groups += (-n_groups) % 4
    ept = n_groups * NB * K
    e_pad = ept * N_SUBCORES
    junk = n

    src = edge_index[0].astype(jnp.int32)
    dst = edge_index[1].astype(jnp.int32)
    pad = e_pad - e
    srcp = jnp.concatenate([src, jnp.zeros((pad,), jnp.int32)])
    dstp = jnp.concatenate([dst, jnp.full((pad,), junk, jnp.int32)])
    idx_all = jnp.stack(
        [srcp.reshape(N_SUBCORES, n_groups, NB, K),
         dstp.reshape(N_SUBCORES, n_groups, NB, K)], axis=3)

    zeros_h = jnp.zeros((n_pad, hidden // N_CORES), jnp.float32)
    zeros_8 = jnp.zeros((n_pad, DEG_W), jnp.float32)
    ones_8 = jnp.ones((K, DEG_W), jnp.float32)

    agg_h = _make_agg(n_pad, hidden, n_groups)
    deg_k = _make_deg(n_pad, n_groups)

    W3p = jnp.pad(W3, ((0, 0), (0, hidden - n_classes)))

    cnt = deg_k(ones_8, idx_all, zeros_8)
    c0 = cnt[0, :n, :]
    c1 = cnt[1, :n, :]

    y1, dinv = _tc_call(_tc1, (
        jax.ShapeDtypeStruct((n_pad, hidden), jnp.float32),
        jax.ShapeDtypeStruct((n, 1), jnp.float32),
    ))(c0, c1, x, W1)

    s = agg_h(y1, idx_all, zeros_h)
    y2 = _tc_call(_tc_mid, jax.ShapeDtypeStruct((n_pad, hidden), jnp.float32))(
        s, y1, dinv, b1.reshape(1, -1), W2)

    s = agg_h(y2, idx_all, zeros_h)
    y3 = _tc_call(_tc_mid, jax.ShapeDtypeStruct((n_pad, hidden), jnp.float32))(
        s, y2, dinv, b2.reshape(1, -1), W3p)

    s = agg_h(y3, idx_all, zeros_h)
    out = _tc_call(_tc_last, jax.ShapeDtypeStruct((n, n_classes), jnp.float32))(
        s, y3, dinv, b3.reshape(1, -1))
    return out


# speedup vs baseline: 1.0004x; 1.0004x over previous
import functools

import jax
import jax.numpy as jnp
from jax import lax
from jax.experimental import pallas as pl
from jax.experimental.pallas import tpu as pltpu
from jax.experimental.pallas import tpu_sc as plsc

N_CORES = 2
N_SUBCORES = 16
K = 128
NB = 10
DEG_W = 8


def _sc_params():
    return pltpu.CompilerParams(use_tc_tiling_on_sc=False)


def _make_agg(n_pad, d_feat, n_groups):
    mesh = plsc.VectorSubcoreMesh(core_axis_name="c", subcore_axis_name="s")
    rows_per_tile = n_pad // N_SUBCORES
    d_half = d_feat // N_CORES

    @functools.partial(
        pl.kernel,
        out_type=jax.ShapeDtypeStruct((n_pad, d_feat), jnp.float32),
        mesh=mesh,
        compiler_params=_sc_params(),
        scratch_types=[
            pltpu.VMEM((2, NB, 2, K), jnp.int32),
            pltpu.VMEM((2, NB, K, d_half), jnp.float32),
            pltpu.VMEM_SHARED((n_pad, d_half), jnp.float32),
            pltpu.VMEM_SHARED((n_pad, d_half), jnp.float32),
            pltpu.SemaphoreType.DMA,
            pltpu.SemaphoreType.DMA,
            pltpu.SemaphoreType.DMA,
        ],
    )
    def agg(y_hbm, idx_hbm, zeros_hbm, out_hbm, idx_v, rows_v, acc_sh, y_sh,
            gsem, isem, ssem):
        cid = lax.axis_index("c")
        sid = lax.axis_index("s")
        r0 = sid * rows_per_tile
        col0 = cid * d_half
        pltpu.sync_copy(zeros_hbm.at[pl.ds(r0, rows_per_tile)],
                        acc_sh.at[pl.ds(r0, rows_per_tile)])
        pltpu.sync_copy(y_hbm.at[pl.ds(r0, rows_per_tile), pl.ds(col0, d_half)],
                        y_sh.at[pl.ds(r0, rows_per_tile)])
        pltpu.sync_copy(idx_hbm.at[sid, 0], idx_v.at[0])
        plsc.subcore_barrier()

        def scatter_descs(parity):
            return [
                pltpu.make_async_copy(rows_v.at[parity, b],
                                      acc_sh.at[idx_v.at[parity, b, 1]],
                                      ssem)
                for b in range(NB)
            ]

        def group_body(parity, g):
            nxt = g + 1
            descs = [
                pltpu.async_copy(y_sh.at[idx_v.at[parity, b, 0]],
                                 rows_v.at[parity, b], gsem)
                for b in range(NB)
            ]

            @pl.when(g > 0)
            def _drain_prev_scatters():
                for d in scatter_descs(1 - parity):
                    d.wait()

            @pl.when(nxt < n_groups)
            def _prefetch():
                pltpu.async_copy(idx_hbm.at[sid, nxt], idx_v.at[1 - parity], isem)

            for b in range(NB):
                descs[b].wait()
                pltpu.async_copy(rows_v.at[parity, b],
                                 acc_sh.at[idx_v.at[parity, b, 1]], ssem,
                                 add=True)

            @pl.when(nxt < n_groups)
            def _drain_prefetch():
                pltpu.make_async_copy(idx_hbm.at[sid, nxt],
                                      idx_v.at[1 - parity], isem).wait()

        def body(gg, carry):
            group_body(0, 2 * gg)
            group_body(1, 2 * gg + 1)
            return carry

        lax.fori_loop(0, n_groups // 2, body, 0)
        for d in scatter_descs(1):
            d.wait()
        plsc.subcore_barrier()
        pltpu.sync_copy(acc_sh.at[pl.ds(r0, rows_per_tile)],
                        out_hbm.at[pl.ds(r0, rows_per_tile), pl.ds(col0, d_half)])

    return agg


def _make_deg(n_pad, n_groups):
    mesh = plsc.VectorSubcoreMesh(core_axis_name="c", subcore_axis_name="s")
    rows_per_tile = n_pad // N_SUBCORES
    half_groups = n_groups // 2

    @functools.partial(
        pl.kernel,
        out_type=jax.ShapeDtypeStruct((N_CORES, n_pad, DEG_W), jnp.float32),
        mesh=mesh,
        compiler_params=_sc_params(),
        scratch_types=[
            pltpu.VMEM((2, NB, 2, K), jnp.int32),
            pltpu.VMEM((K, DEG_W), jnp.float32),
            pltpu.VMEM_SHARED((n_pad, DEG_W), jnp.float32),
            pltpu.SemaphoreType.DMA,
            pltpu.SemaphoreType.DMA,
        ],
    )
    def deg(ones_hbm, idx_hbm, zeros_hbm, out_hbm, idx_v, ones_v, acc_sh, isem,
            ssem):
        cid = lax.axis_index("c")
        sid = lax.axis_index("s")
        g0 = cid * half_groups
        r0 = sid * rows_per_tile
        pltpu.sync_copy(zeros_hbm.at[pl.ds(r0, rows_per_tile)],
                        acc_sh.at[pl.ds(r0, rows_per_tile)])
        pltpu.sync_copy(ones_hbm, ones_v)
        pltpu.sync_copy(idx_hbm.at[sid, g0], idx_v.at[0])
        plsc.subcore_barrier()

        def scatter_descs(parity):
            return [
                pltpu.make_async_copy(ones_v, acc_sh.at[idx_v.at[parity, b, 1]],
                                      ssem)
                for b in range(NB)
            ]

        def group_body(parity, g):
            nxt = g + 1

            @pl.when(g > 0)
            def _drain_prev_scatters():
                for d in scatter_descs(1 - parity):
                    d.wait()

            @pl.when(nxt < half_groups)
            def _prefetch():
                pltpu.async_copy(idx_hbm.at[sid, g0 + nxt], idx_v.at[1 - parity],
                                 isem)

            for b in range(NB):
                pltpu.async_copy(ones_v, acc_sh.at[idx_v.at[parity, b, 1]],
                                 ssem, add=True)

            @pl.when(nxt < half_groups)
            def _drain_prefetch():
                pltpu.make_async_copy(idx_hbm.at[sid, g0 + nxt],
                                      idx_v.at[1 - parity], isem).wait()

        def body(gg, carry):
            group_body(0, 2 * gg)
            group_body(1, 2 * gg + 1)
            return carry

        lax.fori_loop(0, half_groups // 2, body, 0)
        for d in scatter_descs(1):
            d.wait()
        plsc.subcore_barrier()
        pltpu.sync_copy(acc_sh.at[pl.ds(r0, rows_per_tile)],
                        out_hbm.at[cid, pl.ds(r0, rows_per_tile)])

    return deg


def _tc_call(body, out_shapes):
    return pl.pallas_call(body, out_shape=out_shapes)


def _tc1(c0_ref, c1_ref, x_ref, w1_ref, y1_ref, dinv_ref):
    n = x_ref.shape[0]
    deg = c0_ref[:, :1] + c1_ref[:, :1] + 1.0
    dinv = lax.rsqrt(deg)
    dinv_ref[...] = dinv
    h = jnp.dot(x_ref[...], w1_ref[...], precision=lax.Precision.HIGHEST,
                preferred_element_type=jnp.float32)
    y1_ref[:n] = h * dinv
    y1_ref[n:] = jnp.zeros_like(y1_ref[n:])


def _tc_mid(s_ref, y_ref, dinv_ref, b_ref, w_ref, out_ref):
    n = dinv_ref.shape[0]
    dinv = dinv_ref[...]
    p = dinv * (s_ref[:n] + y_ref[:n]) + b_ref[...]
    h = jnp.maximum(p, 0.0)
    out_ref[:n] = jnp.dot(h, w_ref[...], precision=lax.Precision.HIGHEST,
                          preferred_element_type=jnp.float32) * dinv
    out_ref[n:] = jnp.zeros_like(out_ref[n:])


def _tc_last(s_ref, y_ref, dinv_ref, b_ref, out_ref):
    n, c = out_ref.shape
    z = dinv_ref[...] * (s_ref[:n, :c] + y_ref[:n, :c]) + b_ref[...]
    z = z - jnp.max(z, axis=1, keepdims=True)
    out_ref[...] = z - jnp.log(jnp.sum(jnp.exp(z), axis=1, keepdims=True))


@jax.jit
def kernel(x, edge_index, W1, b1, W2, b2, W3, b3):
    n = x.shape[0]
    e = edge_index.shape[1]
    n_classes = W3.shape[1]
    hidden = W1.shape[1]

    n_pad = ((n + 1 + 8 * N_SUBCORES - 1) // (8 * N_SUBCORES)) * (8 * N_SUBCORES)
    group_e = N_SUBCORES * K * NB
    n_groups = (e + group_e - 1) // group_e
    n_groups += (-n_groups) % 4
    ept = n_groups * NB * K
    e_pad = ept * N_SUBCORES
    junk = n

    src = edge_index[0].astype(jnp.int32)
    dst = edge_index[1].astype(jnp.int32)
    pad = e_pad - e
    srcp = jnp.concatenate([src, jnp.zeros((pad,), jnp.int32)])
    dstp = jnp.concatenate([dst, jnp.full((pad,), junk, jnp.int32)])
    idx_all = jnp.stack(
        [srcp.reshape(N_SUBCORES, n_groups, NB, K),
         dstp.reshape(N_SUBCORES, n_groups, NB, K)], axis=3)

    zeros_h = jnp.zeros((n_pad, hidden // N_CORES), jnp.float32)
    zeros_8 = jnp.zeros((n_pad, DEG_W), jnp.float32)
    ones_8 = jnp.ones((K, DEG_W), jnp.float32)

    agg_h = _make_agg(n_pad, hidden, n_groups)
    deg_k = _make_deg(n_pad, n_groups)

    W3p = jnp.pad(W3, ((0, 0), (0, hidden - n_classes)))

    cnt = deg_k(ones_8, idx_all, zeros_8)
    c0 = cnt[0, :n, :]
    c1 = cnt[1, :n, :]

    y1, dinv = _tc_call(_tc1, (
        jax.ShapeDtypeStruct((n_pad, hidden), jnp.float32),
        jax.ShapeDtypeStruct((n, 1), jnp.float32),
    ))(c0, c1, x, W1)

    s = agg_h(y1, idx_all, zeros_h)
    y2 = _tc_call(_tc_mid, jax.ShapeDtypeStruct((n_pad, hidden), jnp.float32))(
        s, y1, dinv, b1.reshape(1, -1), W2)

    s = agg_h(y2, idx_all, zeros_h)
    y3 = _tc_call(_tc_mid, jax.ShapeDtypeStruct((n_pad, hidden), jnp.float32))(
        s, y2, dinv, b2.reshape(1, -1), W3p)

    s = agg_h(y3, idx_all, zeros_h)
    out = _tc_call(_tc_last, jax.ShapeDtypeStruct((n, n_classes), jnp.float32))(
        s, y3, dinv, b3.reshape(1, -1))
    return out
